# baseline (speedup 1.0000x reference)
import jax
import jax.numpy as jnp
from jax.experimental import pallas as pl
from jax.experimental.pallas import tpu as pltpu
from jax.experimental.pallas import tpu_sc as plsc

_WINDOW = 128


def kernel(token_ids, weight):
    b, s = token_ids.shape
    n = b * s
    v, d = weight.shape
    flat_ids = token_ids.reshape(n).astype(jnp.int32)
    pair_ids = (flat_ids // 2).reshape(1, n)
    w2 = weight.reshape(v // 2, 2 * d)
    mesh = plsc.VectorSubcoreMesh(core_axis_name="c", subcore_axis_name="s")

    @pl.kernel(
        out_type=jax.ShapeDtypeStruct((n, 2 * d), weight.dtype),
        mesh=mesh,
    )
    def gather_kernel(w_hbm, i_hbm, o_hbm):
        def body(i_vmem, o_vmem):
            pltpu.sync_copy(w_hbm.at[i_vmem.at[0]], o_vmem)

        pltpu.emit_pipeline(
            body,
            grid=(n // _WINDOW,),
            in_specs=[pl.BlockSpec((1, _WINDOW), index_map=lambda i: (0, i))],
            out_specs=[pl.BlockSpec((_WINDOW, 2 * d), index_map=lambda i: (i, 0))],
            core_axis_name=("c", "s"),
            dimension_semantics=(pltpu.PARALLEL,),
        )(i_hbm, o_hbm)

    pairs = gather_kernel(w2, pair_ids)
    parity = (flat_ids % 2)[:, None]
    out = jnp.where(parity == 0, pairs[:, :d], pairs[:, d:])
    return out.reshape(b, s, d)


# speedup vs baseline: 1.3126x; 1.3126x over previous
import jax
import jax.numpy as jnp
from jax.experimental import pallas as pl
from jax.experimental.pallas import tpu as pltpu
from jax.experimental.pallas import tpu_sc as plsc

_WINDOW = 128
_TC1_BLOCK_V = 4096
_TC2_BLOCK_B = 512


def _cdiv(a, b):
    return (a + b - 1) // b


def kernel(token_ids, weight):
    b, s = token_ids.shape
    n = b * s
    v, d = weight.shape
    w_t = weight.T
    ids_t = token_ids.T
    flat_ids = ids_t.reshape(1, n).astype(jnp.int32)

    def relayout_body(wt_ref, o_ref):
        t = jnp.transpose(wt_ref[...], (1, 0))
        o_ref[...] = jnp.concatenate([t, t], axis=1)

    w2 = pl.pallas_call(
        relayout_body,
        grid=(_cdiv(v, _TC1_BLOCK_V),),
        in_specs=[pl.BlockSpec((d, _TC1_BLOCK_V), lambda i: (0, i))],
        out_specs=pl.BlockSpec((_TC1_BLOCK_V, 2 * d), lambda i: (i, 0)),
        out_shape=jax.ShapeDtypeStruct((v, 2 * d), jnp.float32),
        compiler_params=pltpu.CompilerParams(
            dimension_semantics=("parallel",)),
    )(w_t)

    mesh = plsc.VectorSubcoreMesh(core_axis_name="c", subcore_axis_name="s")

    @pl.kernel(
        out_type=jax.ShapeDtypeStruct((n, 2 * d), jnp.float32),
        mesh=mesh,
    )
    def gather_kernel(w_hbm, i_hbm, o_hbm):
        def body(i_vmem, o_vmem):
            pltpu.sync_copy(w_hbm.at[i_vmem.at[0]], o_vmem)

        pltpu.emit_pipeline(
            body,
            grid=(n // _WINDOW,),
            in_specs=[pl.BlockSpec((1, _WINDOW), index_map=lambda i: (0, i))],
            out_specs=[pl.BlockSpec((_WINDOW, 2 * d), index_map=lambda i: (i, 0))],
            core_axis_name=("c", "s"),
            dimension_semantics=(pltpu.PARALLEL,),
        )(i_hbm, o_hbm)

    g = gather_kernel(w2, flat_ids)

    def select_body(g_ref, o_ref):
        o_ref[...] = jnp.transpose(g_ref[...][:, :d], (1, 0))[None]

    n_bblk = b // _TC2_BLOCK_B

    out_t = pl.pallas_call(
        select_body,
        grid=(s, n_bblk),
        in_specs=[pl.BlockSpec((_TC2_BLOCK_B, 2 * d),
                               lambda i, j: (i * n_bblk + j, 0))],
        out_specs=pl.BlockSpec((1, d, _TC2_BLOCK_B), lambda i, j: (i, 0, j)),
        out_shape=jax.ShapeDtypeStruct((s, d, b), jnp.float32),
        compiler_params=pltpu.CompilerParams(
            dimension_semantics=("parallel", "parallel")),
    )(g)

    return jnp.transpose(out_t, (2, 0, 1))
